# baseline (speedup 1.0000x reference)
import functools

import jax
import jax.numpy as jnp
from jax import lax
from jax.experimental import pallas as pl
from jax.experimental.pallas import tpu as pltpu
from jax.experimental.pallas import tpu_sc as plsc

NC = 2
NS = 16
NW = NC * NS
CHUNK = 128
BG = 16


def _make_sc_gather(B, D, n_chunks, b_per_w):
    mesh = plsc.VectorSubcoreMesh(core_axis_name="c", subcore_axis_name="s")

    @functools.partial(
        pl.kernel,
        mesh=mesh,
        compiler_params=pltpu.CompilerParams(use_tc_tiling_on_sc=False),
        out_type=(
            jax.ShapeDtypeStruct((B, D), jnp.float32),
            jax.ShapeDtypeStruct((B, D), jnp.float32),
            jax.ShapeDtypeStruct((B, BG), jnp.float32),
        ),
        scratch_types=[
            pltpu.VMEM((n_chunks, CHUNK), jnp.int32),
            pltpu.VMEM((n_chunks, CHUNK), jnp.int32),
            pltpu.VMEM((n_chunks, CHUNK), jnp.int32),
            pltpu.VMEM((b_per_w, D), jnp.float32),
            pltpu.VMEM((b_per_w, D), jnp.float32),
            pltpu.VMEM((b_per_w, BG), jnp.float32),
            pltpu.SemaphoreType.DMA,
        ],
    )
    def sc_gather(stu_hbm, exer_hbm, exer16_hbm, theta_hbm, a_hbm, b16_hbm,
                  theta_out, a_out, b_out,
                  sidx_v, eidx_v, bidx_v, theta_v, a_v, b_v, sem):
        wid = lax.axis_index("s") * NC + lax.axis_index("c")
        base = wid * b_per_w
        row0 = wid * n_chunks
        pltpu.sync_copy(stu_hbm.at[pl.ds(row0, n_chunks)], sidx_v)
        pltpu.sync_copy(exer_hbm.at[pl.ds(row0, n_chunks)], eidx_v)
        pltpu.sync_copy(exer16_hbm.at[pl.ds(row0, n_chunks)], bidx_v)
        copies = []
        for j in range(n_chunks):
            dst = pl.ds(j * CHUNK, CHUNK)
            copies.append(pltpu.async_copy(
                theta_hbm.at[sidx_v.at[j]], theta_v.at[dst], sem))
            copies.append(pltpu.async_copy(
                a_hbm.at[eidx_v.at[j]], a_v.at[dst], sem))
            copies.append(pltpu.async_copy(
                b16_hbm.at[bidx_v.at[j]], b_v.at[dst], sem))
        for c in copies:
            c.wait()
        pltpu.sync_copy(theta_v, theta_out.at[pl.ds(base, b_per_w)])
        pltpu.sync_copy(a_v, a_out.at[pl.ds(base, b_per_w)])
        pltpu.sync_copy(b_v, b_out.at[pl.ds(base, b_per_w)])

    return sc_gather


def _score_body(theta_ref, a_ref, b16_ref, bmod_ref, out_ref):
    a = 1.0 / (1.0 + jnp.exp(-a_ref[...]))
    dot = jnp.sum(a * theta_ref[...], axis=-1)
    sel = lax.broadcasted_iota(jnp.int32, b16_ref.shape, 1) == bmod_ref[...]
    b = jnp.sum(jnp.where(sel, b16_ref[...], 0.0), axis=-1)
    logits = dot - b
    out_ref[...] = 1.0 / (1.0 + jnp.exp(-logits))


def kernel(stu_id, input_exercise, theta_w, a_w, b_w):
    B = stu_id.shape[0]
    D = theta_w.shape[1]
    b_per_w = B // NW
    n_chunks = b_per_w // CHUNK
    stu = stu_id.astype(jnp.int32)
    exer = input_exercise.astype(jnp.int32)
    stu2 = stu.reshape(B // CHUNK, CHUNK)
    exer2 = exer.reshape(B // CHUNK, CHUNK)
    exer16 = (exer2 // BG)

    nb = b_w.shape[0]
    pad = (-nb) % BG
    b_flat = b_w.reshape(nb)
    if pad:
        b_flat = jnp.pad(b_flat, (0, pad))
    b16 = b_flat.reshape((nb + pad) // BG, BG)

    sc_gather = _make_sc_gather(B, D, n_chunks, b_per_w)
    theta_rows, a_rows, b16_rows = sc_gather(
        stu2, exer2, exer16, theta_w, a_w, b16)

    bmod = (exer % BG).reshape(B, 1)
    out = pl.pallas_call(
        _score_body,
        out_shape=jax.ShapeDtypeStruct((B,), jnp.float32),
    )(theta_rows, a_rows, b16_rows, bmod)
    return out


# speedup vs baseline: 7.3392x; 7.3392x over previous
import functools

import jax
import jax.numpy as jnp
from jax import lax
from jax.experimental import pallas as pl
from jax.experimental.pallas import tpu as pltpu
from jax.experimental.pallas import tpu_sc as plsc

NC = 2
NS = 16
NW = NC * NS
GW = 16
D = 32
SL = 8


def _make_sc_gather(B, b_per_w, n_groups):
    mesh = plsc.VectorSubcoreMesh(core_axis_name="c", subcore_axis_name="s")

    @functools.partial(
        pl.kernel,
        mesh=mesh,
        compiler_params=pltpu.CompilerParams(
            use_tc_tiling_on_sc=True, needs_layout_passes=False),
        out_type=(
            jax.ShapeDtypeStruct((D, B), jnp.float32),
            jax.ShapeDtypeStruct((D, B), jnp.float32),
            jax.ShapeDtypeStruct((B,), jnp.float32),
        ),
        scratch_types=[
            pltpu.VMEM((b_per_w // 128, 128), jnp.int32),
            pltpu.VMEM((b_per_w // 128, 128), jnp.int32),
            pltpu.VMEM((2, 4, SL, 128), jnp.float32),
            pltpu.VMEM((2, 4, SL, 128), jnp.float32),
            pltpu.VMEM((2, 4, SL, 128), jnp.float32),
            pltpu.VMEM((2, 4, SL, 128), jnp.float32),
            pltpu.VMEM((256,), jnp.float32),
            pltpu.VMEM((256,), jnp.float32),
            pltpu.VMEM((D, b_per_w), jnp.float32),
            pltpu.VMEM((D, b_per_w), jnp.float32),
            pltpu.VMEM((b_per_w,), jnp.float32),
            pltpu.SemaphoreType.DMA,
            pltpu.SemaphoreType.DMA,
        ],
    )
    def sc_gather(stu_hbm, exer_hbm, th3_hbm, a3_hbm, b1_hbm,
                  th_out, a_out, b_out,
                  sidx, eidx, thA, thB, aA, aB, bA, bB,
                  th_sel, a_sel, b_sel, semA, semB):
        wid = lax.axis_index("s") * NC + lax.axis_index("c")
        base = wid * b_per_w
        n_irows = b_per_w // 128
        grow = wid * n_irows
        pltpu.sync_copy(stu_hbm.at[pl.ds(grow, n_irows)], sidx)
        pltpu.sync_copy(exer_hbm.at[pl.ds(grow, n_irows)], eidx)

        lanes = lax.broadcasted_iota(jnp.int32, (GW,), 0)

        def load_idx(ref, g):
            return ref[g // 8, pl.ds((g % 8) * GW, GW)]

        def issue_group(g, th_buf, a_buf, b_buf, sem):
            vs = load_idx(sidx, g)
            ve = load_idx(eidx, g)

            @pl.loop(0, GW)
            def _(j):
                m = lanes == j
                sj = jnp.sum(jnp.where(m, vs, 0))
                ej = jnp.sum(jnp.where(m, ve, 0))
                s_start = (sj // GW) * GW
                e_start = (ej // GW) * GW
                half, slot = j // 8, (j % 8) * GW
                pltpu.async_copy(
                    th3_hbm.at[:, :, pl.ds(s_start, GW)],
                    th_buf.at[half, :, :, pl.ds(slot, GW)], sem)
                pltpu.async_copy(
                    a3_hbm.at[:, :, pl.ds(e_start, GW)],
                    a_buf.at[half, :, :, pl.ds(slot, GW)], sem)
                pltpu.async_copy(
                    b1_hbm.at[pl.ds(e_start, GW)],
                    b_buf.at[pl.ds(j * GW, GW)], sem)

        def drain_group(th_buf, a_buf, b_buf, sem):
            dummy3 = th3_hbm.at[:, :, pl.ds(0, 128)]
            for buf in (th_buf, a_buf):
                pltpu.make_async_copy(dummy3, buf.at[0], sem).wait()
                pltpu.make_async_copy(dummy3, buf.at[1], sem).wait()
            pltpu.make_async_copy(b1_hbm.at[pl.ds(0, 256)], b_buf, sem).wait()

        def select_group(g, th_buf, a_buf, b_buf):
            vs = load_idx(sidx, g)
            ve = load_idx(eidx, g)
            half_v = lanes // 8
            s_lane = (lanes % 8) * GW + lax.rem(vs, GW)
            e_lane = (lanes % 8) * GW + lax.rem(ve, GW)
            goff = g * GW

            @pl.loop(0, D)
            def _(d):
                d0 = jnp.full((GW,), 0, jnp.int32) + d // SL
                d1 = jnp.full((GW,), 0, jnp.int32) + lax.rem(d, SL)
                th_sel[d, pl.ds(goff, GW)] = plsc.load_gather(
                    th_buf, [half_v, d0, d1, s_lane])
                a_sel[d, pl.ds(goff, GW)] = plsc.load_gather(
                    a_buf, [half_v, d0, d1, e_lane])
            b_sel[pl.ds(goff, GW)] = plsc.load_gather(
                b_buf, [lanes * GW + lax.rem(ve, GW)])

        issue_group(0, thA, aA, bA, semA)

        @pl.loop(0, (n_groups - 2) // 2)
        def _(i):
            g = i * 2
            issue_group(g + 1, thB, aB, bB, semB)
            drain_group(thA, aA, bA, semA)
            select_group(g, thA, aA, bA)
            issue_group(g + 2, thA, aA, bA, semA)
            drain_group(thB, aB, bB, semB)
            select_group(g + 1, thB, aB, bB)

        issue_group(n_groups - 1, thB, aB, bB, semB)
        drain_group(thA, aA, bA, semA)
        select_group(n_groups - 2, thA, aA, bA)
        drain_group(thB, aB, bB, semB)
        select_group(n_groups - 1, thB, aB, bB)

        pltpu.sync_copy(th_sel, th_out.at[:, pl.ds(base, b_per_w)])
        pltpu.sync_copy(a_sel, a_out.at[:, pl.ds(base, b_per_w)])
        pltpu.sync_copy(b_sel, b_out.at[pl.ds(base, b_per_w)])

    return sc_gather


def _score_body(th_ref, a_ref, b_ref, out_ref):
    a = 1.0 / (1.0 + jnp.exp(-a_ref[...]))
    logits = jnp.sum(a * th_ref[...], axis=0) - b_ref[...]
    out_ref[...] = 1.0 / (1.0 + jnp.exp(-logits))


def kernel(stu_id, input_exercise, theta_w, a_w, b_w):
    B = stu_id.shape[0]
    b_per_w = B // NW
    n_groups = b_per_w // GW
    stu2 = stu_id.astype(jnp.int32).reshape(B // 128, 128)
    exer2 = input_exercise.astype(jnp.int32).reshape(B // 128, 128)

    th3 = jnp.transpose(theta_w).reshape(4, SL, theta_w.shape[0])
    a3 = jnp.transpose(a_w).reshape(4, SL, a_w.shape[0])
    b1 = b_w.reshape(b_w.shape[0])

    sc_gather = _make_sc_gather(B, b_per_w, n_groups)
    th_g, a_g, b_g = sc_gather(stu2, exer2, th3, a3, b1)

    out = pl.pallas_call(
        _score_body,
        out_shape=jax.ShapeDtypeStruct((B,), jnp.float32),
    )(th_g, a_g, b_g)
    return out


# speedup vs baseline: 8.2177x; 1.1197x over previous
import functools

import jax
import jax.numpy as jnp
from jax import lax
from jax.experimental import pallas as pl
from jax.experimental.pallas import tpu as pltpu
from jax.experimental.pallas import tpu_sc as plsc

NC = 2
NS = 16
NW = NC * NS
GW = 16
D = 32
SL = 8


def _make_sc_kernel(B, b_per_w, n_groups):
    mesh = plsc.VectorSubcoreMesh(core_axis_name="c", subcore_axis_name="s")

    @functools.partial(
        pl.kernel,
        mesh=mesh,
        compiler_params=pltpu.CompilerParams(
            use_tc_tiling_on_sc=True, needs_layout_passes=False),
        out_type=jax.ShapeDtypeStruct((B,), jnp.float32),
        scratch_types=[
            pltpu.VMEM((b_per_w // 128, 128), jnp.int32),
            pltpu.VMEM((b_per_w // 128, 128), jnp.int32),
            pltpu.VMEM((2, 4, SL, 128), jnp.float32),
            pltpu.VMEM((2, 4, SL, 128), jnp.float32),
            pltpu.VMEM((2, 4, SL, 128), jnp.float32),
            pltpu.VMEM((2, 4, SL, 128), jnp.float32),
            pltpu.VMEM((256,), jnp.float32),
            pltpu.VMEM((256,), jnp.float32),
            pltpu.VMEM((b_per_w,), jnp.float32),
            pltpu.SemaphoreType.DMA,
            pltpu.SemaphoreType.DMA,
        ],
    )
    def sc_kernel(stu_hbm, exer_hbm, th3_hbm, a3_hbm, b1_hbm, out_hbm,
                  sidx, eidx, thA, thB, aA, aB, bA, bB, res, semA, semB):
        wid = lax.axis_index("s") * NC + lax.axis_index("c")
        base = wid * b_per_w
        n_irows = b_per_w // 128
        grow = wid * n_irows
        pltpu.sync_copy(stu_hbm.at[pl.ds(grow, n_irows)], sidx)
        pltpu.sync_copy(exer_hbm.at[pl.ds(grow, n_irows)], eidx)

        lanes = lax.broadcasted_iota(jnp.int32, (GW,), 0)

        def load_idx(ref, g):
            return ref[g // 8, pl.ds((g % 8) * GW, GW)]

        def issue_group(g, th_buf, a_buf, b_buf, sem):
            vs = load_idx(sidx, g)
            ve = load_idx(eidx, g)

            @pl.loop(0, GW)
            def _(j):
                m = lanes == j
                sj = jnp.sum(jnp.where(m, vs, 0))
                ej = jnp.sum(jnp.where(m, ve, 0))
                s_start = (sj // GW) * GW
                e_start = (ej // GW) * GW
                half, slot = j // 8, (j % 8) * GW
                pltpu.async_copy(
                    th3_hbm.at[:, :, pl.ds(s_start, GW)],
                    th_buf.at[half, :, :, pl.ds(slot, GW)], sem)
                pltpu.async_copy(
                    a3_hbm.at[:, :, pl.ds(e_start, GW)],
                    a_buf.at[half, :, :, pl.ds(slot, GW)], sem)
                pltpu.async_copy(
                    b1_hbm.at[pl.ds(e_start, GW)],
                    b_buf.at[pl.ds(j * GW, GW)], sem)

        def drain_group(th_buf, a_buf, b_buf, sem):
            dummy3 = th3_hbm.at[:, :, pl.ds(0, 128)]
            for buf in (th_buf, a_buf):
                pltpu.make_async_copy(dummy3, buf.at[0], sem).wait()
                pltpu.make_async_copy(dummy3, buf.at[1], sem).wait()
            pltpu.make_async_copy(b1_hbm.at[pl.ds(0, 256)], b_buf, sem).wait()

        def compute_group(g, th_buf, a_buf, b_buf):
            vs = load_idx(sidx, g)
            ve = load_idx(eidx, g)
            half_v = lanes // 8
            s_lane = (lanes % 8) * GW + lax.rem(vs, GW)
            e_lane = (lanes % 8) * GW + lax.rem(ve, GW)
            zero_v = jnp.zeros((GW,), jnp.int32)

            def dbody(d, acc):
                d0 = zero_v + d // SL
                d1 = zero_v + lax.rem(d, SL)
                th_v = plsc.load_gather(th_buf, [half_v, d0, d1, s_lane])
                a_v = plsc.load_gather(a_buf, [half_v, d0, d1, e_lane])
                asig = 1.0 / (1.0 + jnp.exp(-a_v))
                return acc + asig * th_v

            acc = lax.fori_loop(0, D, dbody, jnp.zeros((GW,), jnp.float32))
            b_v = plsc.load_gather(b_buf, [lanes * GW + lax.rem(ve, GW)])
            logit = acc - b_v
            res[pl.ds(g * GW, GW)] = 1.0 / (1.0 + jnp.exp(-logit))

        issue_group(0, thA, aA, bA, semA)

        @pl.loop(0, (n_groups - 2) // 2)
        def _(i):
            g = i * 2
            issue_group(g + 1, thB, aB, bB, semB)
            drain_group(thA, aA, bA, semA)
            compute_group(g, thA, aA, bA)
            issue_group(g + 2, thA, aA, bA, semA)
            drain_group(thB, aB, bB, semB)
            compute_group(g + 1, thB, aB, bB)

        issue_group(n_groups - 1, thB, aB, bB, semB)
        drain_group(thA, aA, bA, semA)
        compute_group(n_groups - 2, thA, aA, bA)
        drain_group(thB, aB, bB, semB)
        compute_group(n_groups - 1, thB, aB, bB)

        pltpu.sync_copy(res, out_hbm.at[pl.ds(base, b_per_w)])

    return sc_kernel


def kernel(stu_id, input_exercise, theta_w, a_w, b_w):
    B = stu_id.shape[0]
    b_per_w = B // NW
    n_groups = b_per_w // GW
    stu2 = stu_id.astype(jnp.int32).reshape(B // 128, 128)
    exer2 = input_exercise.astype(jnp.int32).reshape(B // 128, 128)

    th3 = jnp.transpose(theta_w).reshape(4, SL, theta_w.shape[0])
    a3 = jnp.transpose(a_w).reshape(4, SL, a_w.shape[0])
    b1 = b_w.reshape(b_w.shape[0])

    sc_kernel = _make_sc_kernel(B, b_per_w, n_groups)
    return sc_kernel(stu2, exer2, th3, a3, b1)
